# Initial kernel scaffold; baseline (speedup 1.0000x reference)
import jax
import jax.numpy as jnp
from jax.experimental import pallas as pl


def kernel(tokens, attention_x, attention_y, attention_y_dense, W1, b1, W2, b2):
    raise NotImplementedError("write your pallas kernel here")


# speedup vs baseline: 1.6636x; 1.6636x over previous
import math

import jax
import jax.numpy as jnp
from jax import lax
from jax.experimental import pallas as pl

_B, _N, _C = 64, 576, 768
_HID = _C // 4
_SPARSE_RATIO = 0.6
_BETA = 0.25
_NUM_KEEP = max(1, math.ceil(_N * _SPARSE_RATIO))
_PAD_SEL = ((_NUM_KEEP + 1 + 7) // 8) * 8


def _norm_row(a):
    a_min = jnp.min(a, axis=-1, keepdims=True)
    a_max = jnp.max(a, axis=-1, keepdims=True)
    return (a - a_min) / (a_max - a_min + 1e-08)


_EXACT = jax.lax.Precision.HIGHEST
_MLP_PREC = jax.lax.Precision.DEFAULT


def _token_sparse_kernel(tok_ref, ax_ref, ay_ref, ayd_ref, w1_ref, b1_ref,
                         w2_ref, b2_ref, sel_ref, mask_ref):
    f32 = jnp.float32
    t = tok_ref[0]

    h = jnp.dot(t, w1_ref[...], preferred_element_type=f32,
                precision=_MLP_PREC) + b1_ref[...]
    h = 0.5 * h * (1.0 - lax.erf(-h * jnp.float32(0.7071067811865476)))
    sp_col = jax.nn.sigmoid(
        jnp.dot(h, w2_ref[...], preferred_element_type=f32,
                precision=_MLP_PREC) + b2_ref[0, 0]
    )

    sp_row = sp_col.T

    s_im = _norm_row(ax_ref[0])
    s_st = _norm_row(ay_ref[0])
    s_dt = _norm_row(ayd_ref[0])
    score_row = ((1.0 - 2.0 * _BETA) * sp_row
                 + _BETA * (s_st + s_dt + 2.0 * s_im))
    score_col = score_row.T

    i_col = lax.broadcasted_iota(jnp.int32, (_N, _N), 0)
    j_row = lax.broadcasted_iota(jnp.int32, (_N, _N), 1)
    beats = (score_col > score_row) | ((score_col == score_row) & (i_col < j_row))
    rank_row = jnp.sum(beats.astype(f32), axis=0, keepdims=True)
    rank_i = rank_row.astype(jnp.int32)

    mask_ref[0] = (rank_i < _NUM_KEEP).astype(f32)

    nk = rank_i >= _NUM_KEEP
    m = jnp.max(jnp.where(nk, score_row, -jnp.inf), axis=-1, keepdims=True)
    e = jnp.where(nk, jnp.exp(score_row - m), 0.0)
    w_row = e / jnp.sum(e, axis=-1, keepdims=True)

    p_iota = lax.broadcasted_iota(jnp.int32, (_PAD_SEL, _N), 0)
    onehot_bf = ((p_iota == rank_i) & (p_iota < _NUM_KEEP)).astype(jnp.bfloat16)
    t_hi = t.astype(jnp.bfloat16)
    r1 = t - t_hi.astype(f32)
    t_mid = r1.astype(jnp.bfloat16)
    t_lo = (r1 - t_mid.astype(f32)).astype(jnp.bfloat16)
    sel = (jnp.dot(onehot_bf, t_hi, preferred_element_type=f32)
           + jnp.dot(onehot_bf, t_mid, preferred_element_type=f32)
           + jnp.dot(onehot_bf, t_lo, preferred_element_type=f32))
    extra = jnp.dot(w_row, t, preferred_element_type=f32,
                    precision=_MLP_PREC)
    row_extra = (lax.broadcasted_iota(jnp.int32, (_PAD_SEL, 1), 0)
                 == _NUM_KEEP).astype(f32)
    sel_ref[0] = sel + row_extra * extra


def kernel(tokens, attention_x, attention_y, attention_y_dense, W1, b1, W2, b2):
    Bv, Nv, Cv = tokens.shape
    ax = attention_x.reshape(Bv, 1, Nv)
    ay = attention_y.reshape(Bv, 1, Nv)
    ayd = attention_y_dense.reshape(Bv, 1, Nv)
    b1r = b1.reshape(1, _HID)
    b2r = b2.reshape(1, 1)

    sel, mask = pl.pallas_call(
        _token_sparse_kernel,
        grid=(Bv,),
        in_specs=[
            pl.BlockSpec((1, Nv, Cv), lambda b: (b, 0, 0)),
            pl.BlockSpec((1, 1, Nv), lambda b: (b, 0, 0)),
            pl.BlockSpec((1, 1, Nv), lambda b: (b, 0, 0)),
            pl.BlockSpec((1, 1, Nv), lambda b: (b, 0, 0)),
            pl.BlockSpec((Cv, _HID), lambda b: (0, 0)),
            pl.BlockSpec((1, _HID), lambda b: (0, 0)),
            pl.BlockSpec((_HID, 1), lambda b: (0, 0)),
            pl.BlockSpec((1, 1), lambda b: (0, 0)),
        ],
        out_specs=[
            pl.BlockSpec((1, _PAD_SEL, Cv), lambda b: (b, 0, 0)),
            pl.BlockSpec((1, 1, Nv), lambda b: (b, 0, 0)),
        ],
        out_shape=[
            jax.ShapeDtypeStruct((Bv, _PAD_SEL, Cv), jnp.float32),
            jax.ShapeDtypeStruct((Bv, 1, Nv), jnp.float32),
        ],
    )(tokens, ax, ay, ayd, W1, b1r, W2, b2r)

    select_tokens = sel[:, :_NUM_KEEP, :]
    extra_token = sel[:, _NUM_KEEP:_NUM_KEEP + 1, :]
    score_mask = mask.reshape(Bv, Nv)
    return (select_tokens, extra_token, score_mask)
